# baseline (speedup 1.0000x reference)
import functools

import jax
import jax.numpy as jnp
from jax import lax
from jax.experimental import pallas as pl
from jax.experimental.pallas import tpu as pltpu
from jax.experimental.pallas import tpu_sc as plsc

N0 = 10000
N1 = 1024
E = 160000
NODE_IN = 128
EDGE_IN = 16
HIDDEN = 16
NUM_CLASS = 40

_NC = 2
_NS = 16
_NW = _NC * _NS

_B = 1024
_EH = 81920
_CHUNK = 320


def _sc_gather(table, idx, chunk):
    R = idx.shape[0]
    D = table.shape[1]
    per = R // _NW
    nch = per // chunk
    mesh = plsc.VectorSubcoreMesh(core_axis_name="c", subcore_axis_name="s")

    @functools.partial(
        pl.kernel,
        mesh=mesh,
        out_type=jax.ShapeDtypeStruct((R, D), jnp.float32),
        scratch_types=[
            pltpu.VMEM((per,), jnp.int32),
            pltpu.VMEM((chunk, D), jnp.float32),
            pltpu.VMEM((chunk, D), jnp.float32),
            pltpu.SemaphoreType.DMA,
            pltpu.SemaphoreType.DMA,
        ],
    )
    def k(table_hbm, idx_hbm, out_hbm, idx_v, rows0, rows1, sem0, sem1):
        wid = lax.axis_index("s") * _NC + lax.axis_index("c")
        base = pl.multiple_of(wid * per, 8)
        pltpu.sync_copy(idx_hbm.at[pl.ds(base, per)], idx_v)
        rows = (rows0, rows1)
        sems = (sem0, sem1)

        def gather_start(j, buf):
            off = pl.multiple_of(j * chunk, 8)
            pltpu.async_copy(
                table_hbm.at[idx_v.at[pl.ds(off, chunk)]], rows[buf], sems[buf]
            )

        def drain(j, buf):
            off = pl.multiple_of(j * chunk, 8)
            pltpu.make_async_copy(
                table_hbm.at[idx_v.at[pl.ds(off, chunk)]], rows[buf], sems[buf]
            ).wait()
            pltpu.sync_copy(rows[buf], out_hbm.at[pl.ds(base + off, chunk)])

        for j in range(nch):
            if j == 0:
                gather_start(0, 0)
            if j + 1 < nch:
                gather_start(j + 1, (j + 1) % 2)
            drain(j, j % 2)

    return k(table, idx)


def _messages(B, ef_ref, hs_ref, WeT_ref, be8_ref):
    ef = ef_ref[...]
    hs = hs_ref[...]
    lane16 = lax.broadcasted_iota(jnp.int32, (B, HIDDEN), 1)
    m = jnp.zeros((B, HIDDEN), jnp.float32)
    for hh in range(8):
        Y = jnp.dot(ef, WeT_ref[:, hh * 256:(hh + 1) * 256],
                    preferred_element_type=jnp.float32)
        A = jnp.maximum(Y + be8_ref[hh:hh + 1, :], 0.0)
        s0 = jnp.sum(A[:, :128] * hs, axis=1, keepdims=True)
        s1 = jnp.sum(A[:, 128:] * hs, axis=1, keepdims=True)
        m = m + jnp.where(lane16 == 2 * hh, s0, 0.0)
        m = m + jnp.where(lane16 == 2 * hh + 1, s1, 0.0)
    return m


def _segsum(dst_ref, m, B):
    dst = dst_ref[0, 0, :]
    rows = lax.broadcasted_iota(jnp.int32, (N1, B), 0)
    oh = jnp.where(rows == dst[None, :], 1.0, 0.0).astype(jnp.bfloat16)
    return jnp.dot(oh, m.astype(jnp.bfloat16),
                   preferred_element_type=jnp.float32)


def _tc_half1_body(nb, B, ef_ref, hs_ref, dst_ref, WeT_ref, be8_ref,
                   out_ref, acc_ref):
    i = pl.program_id(0)
    contrib = _segsum(dst_ref, _messages(B, ef_ref, hs_ref, WeT_ref, be8_ref),
                      B)

    @pl.when(i == 0)
    def _():
        acc_ref[...] = contrib

    @pl.when(i > 0)
    def _():
        acc_ref[...] = acc_ref[...] + contrib

    @pl.when(i == nb - 1)
    def _():
        out_ref[...] = acc_ref[...]


def _tc_half2_body(nb, B, ef_ref, hs_ref, dst_ref, p1_ref, selfh_ref,
                   WeT_ref, be8_ref, WnT_ref, bn_ref, WfcT_ref, bfc_ref,
                   out_ref, acc_ref):
    i = pl.program_id(0)
    contrib = _segsum(dst_ref, _messages(B, ef_ref, hs_ref, WeT_ref, be8_ref),
                      B)

    @pl.when(i == 0)
    def _():
        acc_ref[...] = contrib

    @pl.when(i > 0)
    def _():
        acc_ref[...] = acc_ref[...] + contrib

    @pl.when(i == nb - 1)
    def _():
        h_dst = acc_ref[...] + p1_ref[...]
        sh = selfh_ref[...]
        z = jnp.dot(sh, WnT_ref[...], preferred_element_type=jnp.float32)
        act = h_dst + jnp.maximum(z + bn_ref[...], 0.0)
        out_ref[...] = (jnp.dot(act, WfcT_ref[...],
                                preferred_element_type=jnp.float32)
                        + bfc_ref[...])


def _whole(shape):
    nd = len(shape)
    return pl.BlockSpec(shape, lambda i: (0,) * nd)


def _tc_half1(ef, h_src, dst3, WeT, be8):
    nb = ef.shape[0] // _B
    return pl.pallas_call(
        functools.partial(_tc_half1_body, nb, _B),
        grid=(nb,),
        in_specs=[
            pl.BlockSpec((_B, EDGE_IN), lambda i: (i, 0)),
            pl.BlockSpec((_B, NODE_IN), lambda i: (i, 0)),
            pl.BlockSpec((1, 1, _B), lambda i: (i, 0, 0)),
            _whole((EDGE_IN, HIDDEN * NODE_IN)),
            _whole((8, 256)),
        ],
        out_specs=_whole((N1, HIDDEN)),
        out_shape=jax.ShapeDtypeStruct((N1, HIDDEN), jnp.float32),
        scratch_shapes=[pltpu.VMEM((N1, HIDDEN), jnp.float32)],
        compiler_params=pltpu.CompilerParams(
            dimension_semantics=("arbitrary",)),
    )(ef, h_src, dst3, WeT, be8)


def _tc_half2(ef, h_src, dst3, p1, self_h, WeT, be8, WnT, bn2, WfcT, bfc2):
    nb = ef.shape[0] // _B
    return pl.pallas_call(
        functools.partial(_tc_half2_body, nb, _B),
        grid=(nb,),
        in_specs=[
            pl.BlockSpec((_B, EDGE_IN), lambda i: (i, 0)),
            pl.BlockSpec((_B, NODE_IN), lambda i: (i, 0)),
            pl.BlockSpec((1, 1, _B), lambda i: (i, 0, 0)),
            _whole((N1, HIDDEN)),
            _whole((N1, NODE_IN)),
            _whole((EDGE_IN, HIDDEN * NODE_IN)),
            _whole((8, 256)),
            _whole((NODE_IN, HIDDEN)),
            _whole((1, HIDDEN)),
            _whole((HIDDEN, NUM_CLASS)),
            _whole((1, NUM_CLASS)),
        ],
        out_specs=_whole((N1, NUM_CLASS)),
        out_shape=jax.ShapeDtypeStruct((N1, NUM_CLASS), jnp.float32),
        scratch_shapes=[pltpu.VMEM((N1, HIDDEN), jnp.float32)],
        compiler_params=pltpu.CompilerParams(
            dimension_semantics=("arbitrary",)),
    )(ef, h_src, dst3, p1, self_h, WeT, be8, WnT, bn2, WfcT, bfc2)


def kernel(node_features, edge_features, edge_src, edge_dst, layer_nid,
           We, be, Wn, bn, Wfc, bfc):
    src = edge_src.astype(jnp.int32)
    dst = edge_dst.astype(jnp.int32)
    nid = layer_nid.astype(jnp.int32)

    idx1 = src[:_EH]
    idx2 = jnp.concatenate([src[_EH:], nid,
                            jnp.zeros((2 * _EH - E - N1,), jnp.int32)])
    g1 = _sc_gather(node_features, idx1, _CHUNK)
    g2 = _sc_gather(node_features, idx2, _CHUNK)
    self_h = g2[E - _EH:E - _EH + N1]

    efb = edge_features.astype(jnp.bfloat16)
    ef2 = jnp.concatenate(
        [efb[_EH:], jnp.zeros((2 * _EH - E, EDGE_IN), jnp.bfloat16)])
    dst1 = dst[:_EH].reshape(_EH // _B, 1, _B)
    dst2 = jnp.concatenate(
        [dst[_EH:], jnp.full((2 * _EH - E,), N1, jnp.int32)]
    ).reshape(_EH // _B, 1, _B)

    WeT = We.T.astype(jnp.bfloat16)
    be8 = be.reshape(8, 256)
    WnT = Wn.T
    bn2 = bn.reshape(1, HIDDEN)
    WfcT = Wfc.T
    bfc2 = bfc.reshape(1, NUM_CLASS)

    p1 = _tc_half1(efb[:_EH], g1, dst1, WeT, be8)
    return _tc_half2(ef2, g2, dst2, p1, self_h, WeT, be8, WnT, bn2,
                     WfcT, bfc2)


# speedup vs baseline: 1.2145x; 1.2145x over previous
import functools

import jax
import jax.numpy as jnp
from jax import lax
from jax.experimental import pallas as pl
from jax.experimental.pallas import tpu as pltpu
from jax.experimental.pallas import tpu_sc as plsc

N0 = 10000
N1 = 1024
E = 160000
NODE_IN = 128
EDGE_IN = 16
HIDDEN = 16
NUM_CLASS = 40

_NC = 2
_NS = 16
_NW = _NC * _NS


def _sc_gather(table, idx, chunk):
    R = idx.shape[0]
    D = table.shape[1]
    per = R // _NW
    nch = per // chunk
    mesh = plsc.VectorSubcoreMesh(core_axis_name="c", subcore_axis_name="s")

    @functools.partial(
        pl.kernel,
        mesh=mesh,
        out_type=jax.ShapeDtypeStruct((R, D), jnp.float32),
        scratch_types=[
            pltpu.VMEM((per,), jnp.int32),
            pltpu.VMEM((chunk, D), jnp.float32),
            pltpu.VMEM((chunk, D), jnp.float32),
            pltpu.SemaphoreType.DMA,
            pltpu.SemaphoreType.DMA,
        ],
    )
    def k(table_hbm, idx_hbm, out_hbm, idx_v, rows0, rows1, sem0, sem1):
        wid = lax.axis_index("s") * _NC + lax.axis_index("c")
        base = pl.multiple_of(wid * per, 8)
        pltpu.sync_copy(idx_hbm.at[pl.ds(base, per)], idx_v)
        rows = (rows0, rows1)
        sems = (sem0, sem1)

        def gather_start(j, buf):
            off = pl.multiple_of(j * chunk, 8)
            pltpu.async_copy(
                table_hbm.at[idx_v.at[pl.ds(off, chunk)]], rows[buf], sems[buf]
            )

        def drain(j, buf):
            off = pl.multiple_of(j * chunk, 8)
            pltpu.make_async_copy(
                table_hbm.at[idx_v.at[pl.ds(off, chunk)]], rows[buf], sems[buf]
            ).wait()
            pltpu.sync_copy(rows[buf], out_hbm.at[pl.ds(base + off, chunk)])

        for j in range(nch):
            if j == 0:
                gather_start(0, 0)
            if j + 1 < nch:
                gather_start(j + 1, (j + 1) % 2)
            drain(j, j % 2)

    return k(table, idx)


def _tc_body(nb, B, ef_ref, hs_ref, dst_ref, selfh_ref, WeT_ref, be8_ref,
             WnT_ref, bn_ref, WfcT_ref, bfc_ref, out_ref, acc_ref):
    i = pl.program_id(0)
    ef = ef_ref[...]
    hs = hs_ref[...]
    lane16 = lax.broadcasted_iota(jnp.int32, (B, HIDDEN), 1)
    m = jnp.zeros((B, HIDDEN), jnp.float32)
    for hh in range(8):
        Y = jnp.dot(ef, WeT_ref[:, hh * 256:(hh + 1) * 256],
                    preferred_element_type=jnp.float32)
        A = jnp.maximum(Y + be8_ref[hh:hh + 1, :], 0.0)
        s0 = jnp.sum(A[:, :128] * hs, axis=1, keepdims=True)
        s1 = jnp.sum(A[:, 128:] * hs, axis=1, keepdims=True)
        m = m + jnp.where(lane16 == 2 * hh, s0, 0.0)
        m = m + jnp.where(lane16 == 2 * hh + 1, s1, 0.0)
    dst = dst_ref[0, 0, :]
    rows = lax.broadcasted_iota(jnp.int32, (N1, B), 0)
    oh = jnp.where(rows == dst[None, :], 1.0, 0.0).astype(jnp.bfloat16)
    contrib = jnp.dot(oh, m.astype(jnp.bfloat16),
                      preferred_element_type=jnp.float32)

    @pl.when(i == 0)
    def _():
        acc_ref[...] = contrib

    @pl.when(i > 0)
    def _():
        acc_ref[...] = acc_ref[...] + contrib

    @pl.when(i == nb - 1)
    def _():
        sh = selfh_ref[...]
        z = jnp.dot(sh, WnT_ref[...], preferred_element_type=jnp.float32)
        act = acc_ref[...] + jnp.maximum(z + bn_ref[...], 0.0)
        out_ref[...] = (jnp.dot(act, WfcT_ref[...],
                                preferred_element_type=jnp.float32)
                        + bfc_ref[...])


def _tc_fused(ef, h_src, dst3, self_h, WeT, be8, WnT, bn2, WfcT, bfc2, B,
              interpret=False):
    nb = ef.shape[0] // B
    return pl.pallas_call(
        functools.partial(_tc_body, nb, B),
        grid=(nb,),
        in_specs=[
            pl.BlockSpec((B, EDGE_IN), lambda i: (i, 0)),
            pl.BlockSpec((B, NODE_IN), lambda i: (i, 0)),
            pl.BlockSpec((1, 1, B), lambda i: (i, 0, 0)),
            pl.BlockSpec((N1, NODE_IN), lambda i: (0, 0)),
            pl.BlockSpec((EDGE_IN, HIDDEN * NODE_IN), lambda i: (0, 0)),
            pl.BlockSpec((8, 256), lambda i: (0, 0)),
            pl.BlockSpec((NODE_IN, HIDDEN), lambda i: (0, 0)),
            pl.BlockSpec((1, HIDDEN), lambda i: (0, 0)),
            pl.BlockSpec((HIDDEN, NUM_CLASS), lambda i: (0, 0)),
            pl.BlockSpec((1, NUM_CLASS), lambda i: (0, 0)),
        ],
        out_specs=pl.BlockSpec((N1, NUM_CLASS), lambda i: (0, 0)),
        out_shape=jax.ShapeDtypeStruct((N1, NUM_CLASS), jnp.float32),
        scratch_shapes=[pltpu.VMEM((N1, HIDDEN), jnp.float32)],
        compiler_params=pltpu.CompilerParams(
            dimension_semantics=("arbitrary",)),
        interpret=interpret,
    )(ef, h_src, dst3, self_h, WeT, be8, WnT, bn2, WfcT, bfc2)


_B = 1000
_CHUNK = 296


def kernel(node_features, edge_features, edge_src, edge_dst, layer_nid,
           We, be, Wn, bn, Wfc, bfc):
    idx = jnp.concatenate([edge_src.astype(jnp.int32),
                           layer_nid.astype(jnp.int32)])
    gathered = _sc_gather(node_features, idx, _CHUNK)
    h_src = gathered[:E]
    self_h = gathered[E:]
    dst3 = edge_dst.astype(jnp.int32).reshape(E // _B, 1, _B)
    WeT = We.T.astype(jnp.bfloat16)
    be8 = be.reshape(8, 256)
    WnT = Wn.T
    bn2 = bn.reshape(1, HIDDEN)
    WfcT = Wfc.T
    bfc2 = bfc.reshape(1, NUM_CLASS)
    return _tc_fused(edge_features.astype(jnp.bfloat16), h_src, dst3, self_h,
                     WeT, be8, WnT, bn2, WfcT, bfc2, _B)


# speedup vs baseline: 1.2829x; 1.0563x over previous
import functools

import jax
import jax.numpy as jnp
from jax import lax
from jax.experimental import pallas as pl
from jax.experimental.pallas import tpu as pltpu
from jax.experimental.pallas import tpu_sc as plsc

N0 = 10000
N1 = 1024
E = 160000
NODE_IN = 128
EDGE_IN = 16
HIDDEN = 16
NUM_CLASS = 40

_NC = 2
_NS = 16
_NW = _NC * _NS


def _sc_gather(table, idx, chunk):
    R = idx.shape[0]
    D = table.shape[1]
    per = R // _NW
    nch = per // chunk
    mesh = plsc.VectorSubcoreMesh(core_axis_name="c", subcore_axis_name="s")

    @functools.partial(
        pl.kernel,
        mesh=mesh,
        out_type=jax.ShapeDtypeStruct((R, D), jnp.float32),
        scratch_types=[
            pltpu.VMEM((per,), jnp.int32),
            pltpu.VMEM((chunk, D), jnp.float32),
            pltpu.VMEM((chunk, D), jnp.float32),
            pltpu.SemaphoreType.DMA,
            pltpu.SemaphoreType.DMA,
        ],
    )
    def k(table_hbm, idx_hbm, out_hbm, idx_v, rows0, rows1, sem0, sem1):
        wid = lax.axis_index("s") * _NC + lax.axis_index("c")
        base = pl.multiple_of(wid * per, 8)
        pltpu.sync_copy(idx_hbm.at[pl.ds(base, per)], idx_v)
        rows = (rows0, rows1)
        sems = (sem0, sem1)

        def gather_start(j, buf):
            off = pl.multiple_of(j * chunk, 8)
            pltpu.async_copy(
                table_hbm.at[idx_v.at[pl.ds(off, chunk)]], rows[buf], sems[buf]
            )

        def drain(j, buf):
            off = pl.multiple_of(j * chunk, 8)
            pltpu.make_async_copy(
                table_hbm.at[idx_v.at[pl.ds(off, chunk)]], rows[buf], sems[buf]
            ).wait()
            pltpu.sync_copy(rows[buf], out_hbm.at[pl.ds(base + off, chunk)])

        for j in range(nch):
            if j == 0:
                gather_start(0, 0)
            if j + 1 < nch:
                gather_start(j + 1, (j + 1) % 2)
            drain(j, j % 2)

    return k(table, idx)


def _tc_body(nb, B, ef_ref, hs_ref, dst_ref, selfh_ref, WeT_ref, be8_ref,
             WnT_ref, bn_ref, WfcT_ref, bfc_ref, out_ref, acc_ref):
    i = pl.program_id(0)
    ef = ef_ref[...]
    hs = hs_ref[...]
    lane16 = lax.broadcasted_iota(jnp.int32, (B, HIDDEN), 1)
    m = jnp.zeros((B, HIDDEN), jnp.float32)
    for hh in range(8):
        Y = jnp.dot(ef, WeT_ref[:, hh * 256:(hh + 1) * 256],
                    preferred_element_type=jnp.float32)
        A = jnp.maximum(Y + be8_ref[hh:hh + 1, :], 0.0)
        s0 = jnp.sum(A[:, :128] * hs, axis=1, keepdims=True)
        s1 = jnp.sum(A[:, 128:] * hs, axis=1, keepdims=True)
        m = m + jnp.where(lane16 == 2 * hh, s0, 0.0)
        m = m + jnp.where(lane16 == 2 * hh + 1, s1, 0.0)
    dst = dst_ref[0, 0, :]
    rows = lax.broadcasted_iota(jnp.int32, (N1, B), 0)
    oh = jnp.where(rows == dst[None, :], 1.0, 0.0).astype(jnp.bfloat16)
    contrib = jnp.dot(oh, m.astype(jnp.bfloat16),
                      preferred_element_type=jnp.float32)

    @pl.when(i == 0)
    def _():
        acc_ref[...] = contrib

    @pl.when(i > 0)
    def _():
        acc_ref[...] = acc_ref[...] + contrib

    @pl.when(i == nb - 1)
    def _():
        sh = selfh_ref[...]
        z = jnp.dot(sh, WnT_ref[...], preferred_element_type=jnp.float32)
        act = acc_ref[...] + jnp.maximum(z + bn_ref[...], 0.0)
        out_ref[...] = (jnp.dot(act, WfcT_ref[...],
                                preferred_element_type=jnp.float32)
                        + bfc_ref[...])


def _tc_fused(ef, h_src, dst3, self_h, WeT, be8, WnT, bn2, WfcT, bfc2, B,
              interpret=False):
    nb = ef.shape[0] // B
    return pl.pallas_call(
        functools.partial(_tc_body, nb, B),
        grid=(nb,),
        in_specs=[
            pl.BlockSpec((B, EDGE_IN), lambda i: (i, 0)),
            pl.BlockSpec((B, NODE_IN), lambda i: (i, 0)),
            pl.BlockSpec((1, 1, B), lambda i: (i, 0, 0)),
            pl.BlockSpec((N1, NODE_IN), lambda i: (0, 0)),
            pl.BlockSpec((EDGE_IN, HIDDEN * NODE_IN), lambda i: (0, 0)),
            pl.BlockSpec((8, 256), lambda i: (0, 0)),
            pl.BlockSpec((NODE_IN, HIDDEN), lambda i: (0, 0)),
            pl.BlockSpec((1, HIDDEN), lambda i: (0, 0)),
            pl.BlockSpec((HIDDEN, NUM_CLASS), lambda i: (0, 0)),
            pl.BlockSpec((1, NUM_CLASS), lambda i: (0, 0)),
        ],
        out_specs=pl.BlockSpec((N1, NUM_CLASS), lambda i: (0, 0)),
        out_shape=jax.ShapeDtypeStruct((N1, NUM_CLASS), jnp.float32),
        scratch_shapes=[pltpu.VMEM((N1, HIDDEN), jnp.float32)],
        compiler_params=pltpu.CompilerParams(
            dimension_semantics=("arbitrary",)),
        interpret=interpret,
    )(ef, h_src, dst3, self_h, WeT, be8, WnT, bn2, WfcT, bfc2)


_B = 2000
_CHUNK = 296


def kernel(node_features, edge_features, edge_src, edge_dst, layer_nid,
           We, be, Wn, bn, Wfc, bfc):
    idx = jnp.concatenate([edge_src.astype(jnp.int32),
                           layer_nid.astype(jnp.int32)])
    gathered = _sc_gather(node_features, idx, _CHUNK)
    h_src = gathered[:E]
    self_h = gathered[E:]
    dst3 = edge_dst.astype(jnp.int32).reshape(E // _B, 1, _B)
    WeT = We.T.astype(jnp.bfloat16)
    be8 = be.reshape(8, 256)
    WnT = Wn.T
    bn2 = bn.reshape(1, HIDDEN)
    WfcT = Wfc.T
    bfc2 = bfc.reshape(1, NUM_CLASS)
    return _tc_fused(edge_features.astype(jnp.bfloat16), h_src, dst3, self_h,
                     WeT, be8, WnT, bn2, WfcT, bfc2, _B)


# speedup vs baseline: 1.3252x; 1.0330x over previous
import functools

import jax
import jax.numpy as jnp
from jax import lax
from jax.experimental import pallas as pl
from jax.experimental.pallas import tpu as pltpu
from jax.experimental.pallas import tpu_sc as plsc

N0 = 10000
N1 = 1024
E = 160000
NODE_IN = 128
EDGE_IN = 16
HIDDEN = 16
NUM_CLASS = 40

_NC = 2
_NS = 16
_NW = _NC * _NS


def _sc_gather(table, idx, chunk):
    R = idx.shape[0]
    D = table.shape[1]
    per = R // _NW
    nch = per // chunk
    mesh = plsc.VectorSubcoreMesh(core_axis_name="c", subcore_axis_name="s")

    @functools.partial(
        pl.kernel,
        mesh=mesh,
        out_type=jax.ShapeDtypeStruct((R, D), jnp.float32),
        scratch_types=[
            pltpu.VMEM((per,), jnp.int32),
            pltpu.VMEM((chunk, D), jnp.float32),
            pltpu.VMEM((chunk, D), jnp.float32),
            pltpu.SemaphoreType.DMA,
            pltpu.SemaphoreType.DMA,
        ],
    )
    def k(table_hbm, idx_hbm, out_hbm, idx_v, rows0, rows1, sem0, sem1):
        wid = lax.axis_index("s") * _NC + lax.axis_index("c")
        base = pl.multiple_of(wid * per, 8)
        pltpu.sync_copy(idx_hbm.at[pl.ds(base, per)], idx_v)
        rows = (rows0, rows1)
        sems = (sem0, sem1)

        def gather_start(j, buf):
            off = pl.multiple_of(j * chunk, 8)
            pltpu.async_copy(
                table_hbm.at[idx_v.at[pl.ds(off, chunk)]], rows[buf], sems[buf]
            )

        def drain(j, buf):
            off = pl.multiple_of(j * chunk, 8)
            pltpu.make_async_copy(
                table_hbm.at[idx_v.at[pl.ds(off, chunk)]], rows[buf], sems[buf]
            ).wait()
            pltpu.sync_copy(rows[buf], out_hbm.at[pl.ds(base + off, chunk)])

        for j in range(nch):
            if j == 0:
                gather_start(0, 0)
            if j + 1 < nch:
                gather_start(j + 1, (j + 1) % 2)
            drain(j, j % 2)

    return k(table, idx)


def _tc_body(nb, B, ef_ref, hs_ref, dst_ref, selfh_ref, WeT_ref, be8_ref,
             WnT_ref, bn_ref, WfcT_ref, bfc_ref, out_ref, acc_ref):
    i = pl.program_id(0)
    ef = ef_ref[...]
    hs = hs_ref[...]
    lane16 = lax.broadcasted_iota(jnp.int32, (B, HIDDEN), 1)
    m = jnp.zeros((B, HIDDEN), jnp.float32)
    for hh in range(8):
        Y = jnp.dot(ef, WeT_ref[:, hh * 256:(hh + 1) * 256],
                    preferred_element_type=jnp.float32)
        A = jnp.maximum(Y + be8_ref[hh:hh + 1, :], 0.0)
        s0 = jnp.sum(A[:, :128] * hs, axis=1, keepdims=True)
        s1 = jnp.sum(A[:, 128:] * hs, axis=1, keepdims=True)
        m = m + jnp.where(lane16 == 2 * hh, s0, 0.0)
        m = m + jnp.where(lane16 == 2 * hh + 1, s1, 0.0)
    dst = dst_ref[0, 0, :]
    rows = lax.broadcasted_iota(jnp.int32, (N1, B), 0)
    oh = jnp.where(rows == dst[None, :], 1.0, 0.0).astype(jnp.bfloat16)
    contrib = jnp.dot(oh, m.astype(jnp.bfloat16),
                      preferred_element_type=jnp.float32)

    @pl.when(i == 0)
    def _():
        acc_ref[...] = contrib

    @pl.when(i > 0)
    def _():
        acc_ref[...] = acc_ref[...] + contrib

    @pl.when(i == nb - 1)
    def _():
        sh = selfh_ref[...]
        z = jnp.dot(sh, WnT_ref[...], preferred_element_type=jnp.float32)
        act = acc_ref[...] + jnp.maximum(z + bn_ref[...], 0.0)
        out_ref[...] = (jnp.dot(act, WfcT_ref[...],
                                preferred_element_type=jnp.float32)
                        + bfc_ref[...])


def _tc_fused(ef, h_src, dst3, self_h, WeT, be8, WnT, bn2, WfcT, bfc2, B,
              interpret=False):
    nb = ef.shape[0] // B
    return pl.pallas_call(
        functools.partial(_tc_body, nb, B),
        grid=(nb,),
        in_specs=[
            pl.BlockSpec((B, EDGE_IN), lambda i: (i, 0)),
            pl.BlockSpec((B, NODE_IN), lambda i: (i, 0)),
            pl.BlockSpec((1, 1, B), lambda i: (i, 0, 0)),
            pl.BlockSpec((N1, NODE_IN), lambda i: (0, 0)),
            pl.BlockSpec((EDGE_IN, HIDDEN * NODE_IN), lambda i: (0, 0)),
            pl.BlockSpec((8, 256), lambda i: (0, 0)),
            pl.BlockSpec((NODE_IN, HIDDEN), lambda i: (0, 0)),
            pl.BlockSpec((1, HIDDEN), lambda i: (0, 0)),
            pl.BlockSpec((HIDDEN, NUM_CLASS), lambda i: (0, 0)),
            pl.BlockSpec((1, NUM_CLASS), lambda i: (0, 0)),
        ],
        out_specs=pl.BlockSpec((N1, NUM_CLASS), lambda i: (0, 0)),
        out_shape=jax.ShapeDtypeStruct((N1, NUM_CLASS), jnp.float32),
        scratch_shapes=[pltpu.VMEM((N1, HIDDEN), jnp.float32)],
        compiler_params=pltpu.CompilerParams(
            dimension_semantics=("arbitrary",)),
        interpret=interpret,
    )(ef, h_src, dst3, self_h, WeT, be8, WnT, bn2, WfcT, bfc2)


_B = 4000
_CHUNK = 296


def kernel(node_features, edge_features, edge_src, edge_dst, layer_nid,
           We, be, Wn, bn, Wfc, bfc):
    idx = jnp.concatenate([edge_src.astype(jnp.int32),
                           layer_nid.astype(jnp.int32)])
    gathered = _sc_gather(node_features, idx, _CHUNK)
    h_src = gathered[:E]
    self_h = gathered[E:]
    dst3 = edge_dst.astype(jnp.int32).reshape(E // _B, 1, _B)
    WeT = We.T.astype(jnp.bfloat16)
    be8 = be.reshape(8, 256)
    WnT = Wn.T
    bn2 = bn.reshape(1, HIDDEN)
    WfcT = Wfc.T
    bfc2 = bfc.reshape(1, NUM_CLASS)
    return _tc_fused(edge_features.astype(jnp.bfloat16), h_src, dst3, self_h,
                     WeT, be8, WnT, bn2, WfcT, bfc2, _B)
